# baseline (speedup 1.0000x reference)
import functools

import jax
import jax.numpy as jnp
from jax import lax
from jax.experimental import pallas as pl
from jax.experimental.pallas import tpu as pltpu
from jax.experimental.pallas import tpu_sc as plsc

B_, S_, D_ = 4, 8192, 1024
NC, NS, L = 2, 16, 16
NW = NC * NS
ROWS_PER_W = S_ // NW
C = 32
CHUNKS = ROWS_PER_W // C
CD = C * D_
UNROLL = 8

_mesh = plsc.VectorSubcoreMesh(core_axis_name="c", subcore_axis_name="s")


@functools.partial(
    pl.kernel,
    mesh=_mesh,
    out_type=jax.ShapeDtypeStruct((B_ * S_ * D_,), jnp.float32),
    scratch_types=[
        pltpu.VMEM((CD,), jnp.float32),
        pltpu.VMEM((CD,), jnp.float32),
    ],
)
def _sc_add(x_hbm, emb_hbm, out_hbm, xbuf, ebuf):
    wid = lax.axis_index("s") * NC + lax.axis_index("c")
    s0 = wid * ROWS_PER_W

    def chunk_body(j, _):
        e_off = (s0 + j * C) * D_
        pltpu.sync_copy(emb_hbm.at[pl.ds(e_off, CD)], ebuf)

        def batch_body(b, _):
            x_off = b * (S_ * D_) + e_off
            pltpu.sync_copy(x_hbm.at[pl.ds(x_off, CD)], xbuf)

            def add_body(i, _):
                base = i * (L * UNROLL)
                for u in range(UNROLL):
                    sl = pl.ds(base + u * L, L)
                    xbuf[sl] = xbuf[sl] + ebuf[sl]
                return 0

            lax.fori_loop(0, CD // (L * UNROLL), add_body, 0)
            pltpu.sync_copy(xbuf, out_hbm.at[pl.ds(x_off, CD)])
            return 0

        lax.fori_loop(0, B_, batch_body, 0)
        return 0

    lax.fori_loop(0, CHUNKS, chunk_body, 0)


def kernel(x, emb_table):
    b, s, d = x.shape
    out = _sc_add(x.reshape(-1), emb_table[:s].reshape(-1))
    return out.reshape(b, s, d)


# speedup vs baseline: 1.1650x; 1.1650x over previous
import functools

import jax
import jax.numpy as jnp
from jax import lax
from jax.experimental import pallas as pl
from jax.experimental.pallas import tpu as pltpu
from jax.experimental.pallas import tpu_sc as plsc

B_, S_, D_ = 4, 8192, 1024
NC, NS, L = 2, 16, 16
NW = NC * NS
ROWS_PER_W = S_ // NW
C = 16
CHUNKS = ROWS_PER_W // C
CD = C * D_
T = CHUNKS * B_
UNROLL = 8

_mesh = plsc.VectorSubcoreMesh(core_axis_name="c", subcore_axis_name="s")


@functools.partial(
    pl.kernel,
    mesh=_mesh,
    out_type=jax.ShapeDtypeStruct((B_ * S_ * D_,), jnp.float32),
    scratch_types=[
        pltpu.VMEM((2 * CD,), jnp.float32),
        pltpu.VMEM((2 * CD,), jnp.float32),
        pltpu.SemaphoreType.DMA,
        pltpu.SemaphoreType.DMA,
        pltpu.SemaphoreType.DMA,
        pltpu.SemaphoreType.DMA,
        pltpu.SemaphoreType.DMA,
        pltpu.SemaphoreType.DMA,
    ],
)
def _sc_add(x_hbm, emb_hbm, out_hbm, xbuf, ebuf,
            xs0, xs1, es0, es1, ss0, ss1):
    xsem = (xs0, xs1)
    esem = (es0, es1)
    ssem = (ss0, ss1)
    wid = lax.axis_index("s") * NC + lax.axis_index("c")
    base = wid * (ROWS_PER_W * D_)

    xloads = [None] * T
    eloads = [None] * CHUNKS
    stores = [None] * T

    def item_offsets(t):
        j, b = divmod(t, B_)
        off = b * (S_ * D_) + base + j * CD
        return j, b, off

    for t in range(T + 1):
        if t < T:
            j, b, off = item_offsets(t)
            if t >= 2:
                stores[t - 2].wait()
            xloads[t] = pltpu.async_copy(
                x_hbm.at[pl.ds(off, CD)],
                xbuf.at[pl.ds((t % 2) * CD, CD)],
                xsem[t % 2])
            if b == 0:
                eloads[j] = pltpu.async_copy(
                    emb_hbm.at[pl.ds(base + j * CD, CD)],
                    ebuf.at[pl.ds((j % 2) * CD, CD)],
                    esem[j % 2])
        if t >= 1:
            tp = t - 1
            j, b, off = item_offsets(tp)
            xloads[tp].wait()
            if b == 0:
                eloads[j].wait()
            xoff = (tp % 2) * CD
            eoff = (j % 2) * CD

            def add_body(i, _, xoff=xoff, eoff=eoff):
                ib = i * (L * UNROLL)
                for u in range(UNROLL):
                    xs = pl.ds(xoff + ib + u * L, L)
                    es = pl.ds(eoff + ib + u * L, L)
                    xbuf[xs] = xbuf[xs] + ebuf[es]
                return 0

            lax.fori_loop(0, CD // (L * UNROLL), add_body, 0)
            stores[tp] = pltpu.async_copy(
                xbuf.at[pl.ds(xoff, CD)],
                out_hbm.at[pl.ds(off, CD)],
                ssem[tp % 2])

    stores[T - 2].wait()
    stores[T - 1].wait()


def kernel(x, emb_table):
    b, s, d = x.shape
    out = _sc_add(x.reshape(-1), emb_table[:s].reshape(-1))
    return out.reshape(b, s, d)


# speedup vs baseline: 1.1667x; 1.0015x over previous
import functools

import jax
import jax.numpy as jnp
from jax import lax
from jax.experimental import pallas as pl
from jax.experimental.pallas import tpu as pltpu
from jax.experimental.pallas import tpu_sc as plsc

B_, S_, D_ = 4, 8192, 1024
NC, NS, L = 2, 16, 16
NW = NC * NS
ROWS_PER_W = S_ // NW
C = 16
CHUNKS = ROWS_PER_W // C
CD = C * D_
T = CHUNKS * B_
UNROLL = 8

_mesh = plsc.VectorSubcoreMesh(core_axis_name="c", subcore_axis_name="s")


@functools.partial(
    pl.kernel,
    mesh=_mesh,
    out_type=jax.ShapeDtypeStruct((B_ * S_ * D_,), jnp.float32),
    scratch_types=[
        pltpu.VMEM((2 * CD,), jnp.float32),
        pltpu.VMEM((2 * CD,), jnp.float32),
        pltpu.SemaphoreType.DMA,
        pltpu.SemaphoreType.DMA,
        pltpu.SemaphoreType.DMA,
        pltpu.SemaphoreType.DMA,
        pltpu.SemaphoreType.DMA,
        pltpu.SemaphoreType.DMA,
    ],
)
def _sc_add(x_hbm, emb_hbm, out_hbm, xbuf, ebuf,
            xs0, xs1, es0, es1, ss0, ss1):
    xsem = (xs0, xs1)
    esem = (es0, es1)
    ssem = (ss0, ss1)
    wid = lax.axis_index("s") * NC + lax.axis_index("c")
    base = wid * (ROWS_PER_W * D_)

    xloads = [None] * T
    eloads = [None] * CHUNKS
    stores = [None] * T

    def item_offsets(t):
        j, b = divmod(t, B_)
        off = b * (S_ * D_) + base + j * CD
        return j, b, off

    for t in range(T + 1):
        if t < T:
            j, b, off = item_offsets(t)
            if t >= 2:
                stores[t - 2].wait()
            xloads[t] = pltpu.async_copy(
                x_hbm.at[pl.ds(off, CD)],
                xbuf.at[pl.ds((t % 2) * CD, CD)],
                xsem[t % 2])
            if b == 0:
                eloads[j] = pltpu.async_copy(
                    emb_hbm.at[pl.ds(base + j * CD, CD)],
                    ebuf.at[pl.ds((j % 2) * CD, CD)],
                    esem[j % 2])
        if t >= 1:
            tp = t - 1
            j, b, off = item_offsets(tp)
            xloads[tp].wait()
            if b == 0:
                eloads[j].wait()
            xoff = (tp % 2) * CD
            eoff = (j % 2) * CD

            @plsc.parallel_loop(0, CD, L, unroll=UNROLL)
            def add_body(i, xoff=xoff, eoff=eoff):
                xs = pl.ds(xoff + i, L)
                es = pl.ds(eoff + i, L)
                xbuf[xs] = xbuf[xs] + ebuf[es]
            stores[tp] = pltpu.async_copy(
                xbuf.at[pl.ds(xoff, CD)],
                out_hbm.at[pl.ds(off, CD)],
                ssem[tp % 2])

    stores[T - 2].wait()
    stores[T - 1].wait()


def kernel(x, emb_table):
    b, s, d = x.shape
    out = _sc_add(x.reshape(-1), emb_table[:s].reshape(-1))
    return out.reshape(b, s, d)
